# baseline (speedup 1.0000x reference)
import jax
import jax.numpy as jnp
from jax.experimental import pallas as pl
from jax.experimental.pallas import tpu as pltpu

D_MODEL = 2048
D_CD = 8192
D_ER = 128
N_EXPERTS = 4096
NUM_KEYS = 64
H = 8
K_PER_HEAD = 2

BD = 256
BE = 512
NEG = -3.0e38


def _routing_body(h_ref, wq_ref, keysT_ref, eidx_ref, gates_ref, q_ref):
    q_ref[...] = jnp.dot(h_ref[...], wq_ref[...].astype(jnp.bfloat16),
                         preferred_element_type=jnp.float32)
    t = h_ref.shape[0]
    idx = jax.lax.broadcasted_iota(jnp.int32, (t, NUM_KEYS), 1)
    m1s, a1s, m2s, a2s = [], [], [], []
    for p in range(2):
        for hh in range(H):
            g = p * H + hh
            qg = q_ref[:, g * 64:(g + 1) * 64].astype(jnp.bfloat16)
            sim = jnp.dot(qg, keysT_ref[hh, p].astype(jnp.bfloat16),
                          preferred_element_type=jnp.float32)
            m1 = jnp.max(sim, axis=1, keepdims=True)
            a1 = jnp.min(jnp.where(sim == m1, idx, NUM_KEYS), axis=1,
                         keepdims=True)
            sim2 = jnp.where(idx == a1, NEG, sim)
            m2 = jnp.max(sim2, axis=1, keepdims=True)
            a2 = jnp.min(jnp.where(sim2 == m2, idx, NUM_KEYS), axis=1,
                         keepdims=True)
            m1s.append(m1); a1s.append(a1); m2s.append(m2); a2s.append(a2)
    xs1 = jnp.concatenate(m1s[:H], 1)
    xs2 = jnp.concatenate(m2s[:H], 1)
    ax1 = jnp.concatenate(a1s[:H], 1)
    ax2 = jnp.concatenate(a2s[:H], 1)
    ys1 = jnp.concatenate(m1s[H:], 1)
    ys2 = jnp.concatenate(m2s[H:], 1)
    ay1 = jnp.concatenate(a1s[H:], 1)
    ay2 = jnp.concatenate(a2s[H:], 1)
    cands = [xs1 + ys1, xs1 + ys2, xs2 + ys1, xs2 + ys2]
    bv, bp = cands[0], jnp.zeros_like(ax1)
    sv, sp = jnp.full_like(bv, NEG), jnp.zeros_like(ax1)
    for pos in range(1, 4):
        cv = cands[pos]
        gt = cv > bv
        gt2 = cv > sv
        sv_n = jnp.where(gt, bv, jnp.where(gt2, cv, sv))
        sp_n = jnp.where(gt, bp, jnp.where(gt2, pos, sp))
        bv = jnp.where(gt, cv, bv)
        bp = jnp.where(gt, pos, bp)
        sv, sp = sv_n, sp_n
    e_best = jnp.where(bp <= 1, ax1, ax2) * NUM_KEYS + \
        jnp.where((bp % 2) == 0, ay1, ay2)
    e_sec = jnp.where(sp <= 1, ax1, ax2) * NUM_KEYS + \
        jnp.where((sp % 2) == 0, ay1, ay2)
    g_best = jax.nn.sigmoid(bv - sv)
    g_sec = jax.nn.sigmoid(sv - bv)
    eidx_ref[...] = jnp.concatenate([e_best, e_sec], 1)
    gates_ref[...] = jnp.concatenate([g_best, g_sec], 1)


def _routing(h_bf16, W_q, keysT):
    t = h_bf16.shape[0]
    return pl.pallas_call(
        _routing_body,
        in_specs=[
            pl.BlockSpec((t, D_MODEL), lambda: (0, 0)),
            pl.BlockSpec((D_MODEL, D_ER * H), lambda: (0, 0)),
            pl.BlockSpec((H, 2, 64, 64), lambda: (0, 0, 0, 0)),
        ],
        out_specs=[
            pl.BlockSpec((t, 2 * H), lambda: (0, 0)),
            pl.BlockSpec((t, 2 * H), lambda: (0, 0)),
        ],
        out_shape=[
            jax.ShapeDtypeStruct((t, 2 * H), jnp.int32),
            jax.ShapeDtypeStruct((t, 2 * H), jnp.float32),
        ],
        scratch_shapes=[pltpu.VMEM((t, D_ER * H), jnp.float32)],
    )(h_bf16, W_q, keysT)


def _expert_body(h_ref, det_ref, ue_ref, gm_ref, out_ref):
    j = pl.program_id(0)

    @pl.when(j == 0)
    def _init():
        out_ref[...] = jnp.zeros_like(out_ref)

    s = jnp.dot(h_ref[...], det_ref[...],
                preferred_element_type=jnp.float32)
    p = (s * jax.nn.sigmoid(s) * gm_ref[...].astype(jnp.float32)
         ).astype(jnp.bfloat16)
    out_ref[...] += jnp.dot(p, ue_ref[...].astype(jnp.bfloat16),
                            preferred_element_type=jnp.float32)


def _experts(h_bf16, down_embed_T, up_embed, gmask):
    t = h_bf16.shape[0]
    nj = N_EXPERTS // BE
    return pl.pallas_call(
        _expert_body,
        grid=(nj,),
        in_specs=[
            pl.BlockSpec((t, D_MODEL), lambda j: (0, 0)),
            pl.BlockSpec((D_MODEL, BE), lambda j: (0, j)),
            pl.BlockSpec((BE, D_MODEL), lambda j: (j, 0)),
            pl.BlockSpec((t, BE), lambda j: (0, j)),
        ],
        out_specs=pl.BlockSpec((t, D_MODEL), lambda j: (0, 0)),
        out_shape=jax.ShapeDtypeStruct((t, D_MODEL), jnp.float32),
        compiler_params=pltpu.CompilerParams(
            dimension_semantics=("arbitrary",),
        ),
    )(h_bf16, down_embed_T, up_embed, gmask)


def _ffn_body(h_ref, wg_ref, bg_ref, wu_ref, bu_ref, wd_ref, bd_ref, out_ref):
    j = pl.program_id(0)

    @pl.when(j == 0)
    def _init():
        out_ref[...] = jnp.broadcast_to(bd_ref[...], out_ref.shape)

    hb = h_ref[...]
    g = jnp.dot(hb, wg_ref[...].astype(jnp.bfloat16),
                preferred_element_type=jnp.float32) + bg_ref[...]
    u = jnp.dot(hb, wu_ref[...].astype(jnp.bfloat16),
                preferred_element_type=jnp.float32) + bu_ref[...]
    gg = (g * jax.nn.sigmoid(g) * u).astype(jnp.bfloat16)
    out_ref[...] += jnp.dot(gg, wd_ref[...].astype(jnp.bfloat16),
                            preferred_element_type=jnp.float32)


def _ffn(h_bf16, W_gate, b_gate, W_up, b_up, W_down, b_down):
    t = h_bf16.shape[0]
    nj = D_CD // BD
    return pl.pallas_call(
        _ffn_body,
        grid=(nj,),
        in_specs=[
            pl.BlockSpec((t, D_MODEL), lambda j: (0, 0)),
            pl.BlockSpec((D_MODEL, BD), lambda j: (0, j)),
            pl.BlockSpec((1, BD), lambda j: (0, j)),
            pl.BlockSpec((D_MODEL, BD), lambda j: (0, j)),
            pl.BlockSpec((1, BD), lambda j: (0, j)),
            pl.BlockSpec((BD, D_MODEL), lambda j: (j, 0)),
            pl.BlockSpec((1, D_MODEL), lambda j: (0, 0)),
        ],
        out_specs=pl.BlockSpec((t, D_MODEL), lambda j: (0, 0)),
        out_shape=jax.ShapeDtypeStruct((t, D_MODEL), jnp.float32),
        compiler_params=pltpu.CompilerParams(
            dimension_semantics=("arbitrary",),
        ),
    )(h_bf16, W_gate, b_gate.reshape(1, -1), W_up, b_up.reshape(1, -1),
      W_down, b_down.reshape(1, -1))


def kernel(hidden_states, W_q, keys_p, down_embed, up_embed, W_gate, b_gate,
           W_up, b_up, W_down, b_down):
    b, t, d = hidden_states.shape
    h = hidden_states.reshape(t, d)
    h_bf = h.astype(jnp.bfloat16)

    keysT = jnp.transpose(keys_p, (0, 2, 3, 1))
    eidx, gates = _routing(h_bf, W_q, keysT)

    out = _ffn(h_bf, W_gate, b_gate, W_up, b_up, W_down, b_down)

    rows = jnp.arange(t, dtype=jnp.int32)[:, None]
    gmask = jnp.zeros((t, N_EXPERTS), jnp.bfloat16).at[
        jnp.broadcast_to(rows, (t, 2 * H)), eidx].add(
            gates.astype(jnp.bfloat16), mode="drop", unique_indices=False)

    experts_states = _experts(h_bf, down_embed.T.astype(jnp.bfloat16),
                              up_embed, gmask)
    return (out + experts_states).reshape(b, t, d)


# speedup vs baseline: 1.4166x; 1.4166x over previous
import jax
import jax.numpy as jnp
from jax.experimental import pallas as pl
from jax.experimental.pallas import tpu as pltpu

D_MODEL = 2048
D_CD = 8192
D_ER = 128
N_EXPERTS = 4096
NUM_KEYS = 64
H = 8
K_PER_HEAD = 2

BD = 256
BE = 512
NEG = -3.0e38


def _routing_body(h_ref, wq_ref, keysT_ref, eidx_ref, gates_ref, q_ref):
    q_ref[...] = jnp.dot(h_ref[...], wq_ref[...].astype(jnp.bfloat16),
                         preferred_element_type=jnp.float32)
    t = h_ref.shape[0]
    idx = jax.lax.broadcasted_iota(jnp.int32, (t, NUM_KEYS), 1)
    m1s, a1s, m2s, a2s = [], [], [], []
    for p in range(2):
        for hh in range(H):
            g = p * H + hh
            qg = q_ref[:, g * 64:(g + 1) * 64].astype(jnp.bfloat16)
            sim = jnp.dot(qg, keysT_ref[hh, p].astype(jnp.bfloat16),
                          preferred_element_type=jnp.float32)
            m1 = jnp.max(sim, axis=1, keepdims=True)
            a1 = jnp.min(jnp.where(sim == m1, idx, NUM_KEYS), axis=1,
                         keepdims=True)
            sim2 = jnp.where(idx == a1, NEG, sim)
            m2 = jnp.max(sim2, axis=1, keepdims=True)
            a2 = jnp.min(jnp.where(sim2 == m2, idx, NUM_KEYS), axis=1,
                         keepdims=True)
            m1s.append(m1); a1s.append(a1); m2s.append(m2); a2s.append(a2)
    xs1 = jnp.concatenate(m1s[:H], 1)
    xs2 = jnp.concatenate(m2s[:H], 1)
    ax1 = jnp.concatenate(a1s[:H], 1)
    ax2 = jnp.concatenate(a2s[:H], 1)
    ys1 = jnp.concatenate(m1s[H:], 1)
    ys2 = jnp.concatenate(m2s[H:], 1)
    ay1 = jnp.concatenate(a1s[H:], 1)
    ay2 = jnp.concatenate(a2s[H:], 1)
    cands = [xs1 + ys1, xs1 + ys2, xs2 + ys1, xs2 + ys2]
    bv, bp = cands[0], jnp.zeros_like(ax1)
    sv, sp = jnp.full_like(bv, NEG), jnp.zeros_like(ax1)
    for pos in range(1, 4):
        cv = cands[pos]
        gt = cv > bv
        gt2 = cv > sv
        sv_n = jnp.where(gt, bv, jnp.where(gt2, cv, sv))
        sp_n = jnp.where(gt, bp, jnp.where(gt2, pos, sp))
        bv = jnp.where(gt, cv, bv)
        bp = jnp.where(gt, pos, bp)
        sv, sp = sv_n, sp_n
    e_best = jnp.where(bp <= 1, ax1, ax2) * NUM_KEYS + \
        jnp.where((bp % 2) == 0, ay1, ay2)
    e_sec = jnp.where(sp <= 1, ax1, ax2) * NUM_KEYS + \
        jnp.where((sp % 2) == 0, ay1, ay2)
    g_best = jax.nn.sigmoid(bv - sv)
    g_sec = jax.nn.sigmoid(sv - bv)
    eidx_ref[...] = jnp.concatenate([e_best, e_sec], 1)
    gates_ref[...] = jnp.concatenate([g_best, g_sec], 1)


def _routing(h_bf16, W_q, keysT):
    t = h_bf16.shape[0]
    return pl.pallas_call(
        _routing_body,
        in_specs=[
            pl.BlockSpec((t, D_MODEL), lambda: (0, 0)),
            pl.BlockSpec((D_MODEL, D_ER * H), lambda: (0, 0)),
            pl.BlockSpec((H, 2, 64, 64), lambda: (0, 0, 0, 0)),
        ],
        out_specs=[
            pl.BlockSpec((t, 2 * H), lambda: (0, 0)),
            pl.BlockSpec((t, 2 * H), lambda: (0, 0)),
        ],
        out_shape=[
            jax.ShapeDtypeStruct((t, 2 * H), jnp.int32),
            jax.ShapeDtypeStruct((t, 2 * H), jnp.float32),
        ],
        scratch_shapes=[pltpu.VMEM((t, D_ER * H), jnp.float32)],
    )(h_bf16, W_q, keysT)


def _expert_body(h_ref, det_ref, ue_ref, gm_ref, out_ref):
    j = pl.program_id(0)

    @pl.when(j == 0)
    def _init():
        out_ref[...] = jnp.zeros_like(out_ref)

    s = jnp.dot(h_ref[...], det_ref[...],
                preferred_element_type=jnp.float32)
    p = (s * jax.nn.sigmoid(s) * gm_ref[...]).astype(jnp.bfloat16)
    out_ref[...] += jnp.dot(p, ue_ref[...].astype(jnp.bfloat16),
                            preferred_element_type=jnp.float32)


def _experts(h_bf16, down_embed_T, up_embed, gmask):
    t = h_bf16.shape[0]
    nj = N_EXPERTS // BE
    return pl.pallas_call(
        _expert_body,
        grid=(nj,),
        in_specs=[
            pl.BlockSpec((t, D_MODEL), lambda j: (0, 0)),
            pl.BlockSpec((D_MODEL, BE), lambda j: (0, j)),
            pl.BlockSpec((BE, D_MODEL), lambda j: (j, 0)),
            pl.BlockSpec((t, BE), lambda j: (0, j)),
        ],
        out_specs=pl.BlockSpec((t, D_MODEL), lambda j: (0, 0)),
        out_shape=jax.ShapeDtypeStruct((t, D_MODEL), jnp.float32),
        compiler_params=pltpu.CompilerParams(
            dimension_semantics=("arbitrary",),
        ),
    )(h_bf16, down_embed_T, up_embed, gmask)


def _ffn_body(h_ref, wg_ref, bg_ref, wu_ref, bu_ref, wd_ref, bd_ref, out_ref):
    j = pl.program_id(0)

    @pl.when(j == 0)
    def _init():
        out_ref[...] = jnp.broadcast_to(bd_ref[...], out_ref.shape)

    hb = h_ref[...]
    g = jnp.dot(hb, wg_ref[...].astype(jnp.bfloat16),
                preferred_element_type=jnp.float32) + bg_ref[...]
    u = jnp.dot(hb, wu_ref[...].astype(jnp.bfloat16),
                preferred_element_type=jnp.float32) + bu_ref[...]
    gg = (g * jax.nn.sigmoid(g) * u).astype(jnp.bfloat16)
    out_ref[...] += jnp.dot(gg, wd_ref[...].astype(jnp.bfloat16),
                            preferred_element_type=jnp.float32)


def _ffn(h_bf16, W_gate, b_gate, W_up, b_up, W_down, b_down):
    t = h_bf16.shape[0]
    nj = D_CD // BD
    return pl.pallas_call(
        _ffn_body,
        grid=(nj,),
        in_specs=[
            pl.BlockSpec((t, D_MODEL), lambda j: (0, 0)),
            pl.BlockSpec((D_MODEL, BD), lambda j: (0, j)),
            pl.BlockSpec((1, BD), lambda j: (0, j)),
            pl.BlockSpec((D_MODEL, BD), lambda j: (0, j)),
            pl.BlockSpec((1, BD), lambda j: (0, j)),
            pl.BlockSpec((BD, D_MODEL), lambda j: (j, 0)),
            pl.BlockSpec((1, D_MODEL), lambda j: (0, 0)),
        ],
        out_specs=pl.BlockSpec((t, D_MODEL), lambda j: (0, 0)),
        out_shape=jax.ShapeDtypeStruct((t, D_MODEL), jnp.float32),
        compiler_params=pltpu.CompilerParams(
            dimension_semantics=("arbitrary",),
        ),
    )(h_bf16, W_gate, b_gate.reshape(1, -1), W_up, b_up.reshape(1, -1),
      W_down, b_down.reshape(1, -1))


def kernel(hidden_states, W_q, keys_p, down_embed, up_embed, W_gate, b_gate,
           W_up, b_up, W_down, b_down):
    b, t, d = hidden_states.shape
    h = hidden_states.reshape(t, d)
    h_bf = h.astype(jnp.bfloat16)

    keysT = jnp.transpose(keys_p, (0, 2, 3, 1))
    eidx, gates = _routing(h_bf, W_q, keysT)

    out = _ffn(h_bf, W_gate, b_gate, W_up, b_up, W_down, b_down)

    rows = jnp.arange(t, dtype=jnp.int32)[:, None]
    gmask = jnp.zeros((t, N_EXPERTS), jnp.float32).at[
        jnp.broadcast_to(rows, (t, 2 * H)), eidx].add(
            gates, mode="drop", unique_indices=False)

    experts_states = _experts(h_bf, down_embed.T.astype(jnp.bfloat16),
                              up_embed, gmask)
    return (out + experts_states).reshape(b, t, d)
